# Initial kernel scaffold; baseline (speedup 1.0000x reference)
import jax
import jax.numpy as jnp
from jax.experimental import pallas as pl


def kernel(x, edge_index, conv1_Wl, conv1_bl, conv1_Wr, conv2_Wl, conv2_bl, conv2_Wr, conv3_Wl, conv3_bl, conv3_Wr, edge_W, edge_b, node_W, node_b, ecls_W, ecls_b):
    raise NotImplementedError("write your pallas kernel here")


# speedup vs baseline: 20.1807x; 20.1807x over previous
import functools

import jax
import jax.numpy as jnp
from jax import lax
from jax.experimental import pallas as pl
from jax.experimental.pallas import tpu as pltpu
from jax.experimental.pallas import tpu_sc as plsc

N = 100000
E = 3200000
H = 16
DE = 2 * H + 2
DEP = 48

NTILES = 32
LSUB = 128
SUBS = 16
CHUNKS = 49
RPT = SUBS * CHUNKS
ROWS = NTILES * RPT
E_PAD = ROWS * LSUB
N_PAD = 100352
ZROWS = N_PAD // 16
BN = 2048
GRID_N = N_PAD // BN

_mesh = plsc.VectorSubcoreMesh(core_axis_name="c", subcore_axis_name="s")
F32 = jnp.float32


ZTOT = 2 * N_PAD
ZPT = ZTOT // 16


@functools.partial(
    pl.kernel,
    out_type=jax.ShapeDtypeStruct((2, ZTOT), F32),
    mesh=_mesh,
    compiler_params=pltpu.CompilerParams(needs_layout_passes=False, use_tc_tiling_on_sc=False),
    scratch_types=[
        pltpu.VMEM((N_PAD,), F32),
        pltpu.VMEM((SUBS, LSUB), jnp.int32),
        pltpu.VMEM((SUBS, LSUB), jnp.int32),
        pltpu.VMEM((LSUB,), jnp.int32),
        pltpu.VMEM((LSUB,), jnp.int32),
        pltpu.VMEM((LSUB,), F32),
        pltpu.VMEM((LSUB,), F32),
        pltpu.VMEM((256,), F32),
        pltpu.VMEM_SHARED((ZTOT,), F32),
    ],
)
def _sc_deg_agg(x_hbm, src_hbm, dst_hbm, out_hbm,
                xv, srcb, dstb, idx1, idx2, ones, xvals, zbuf, acc):
    c = lax.axis_index("c")
    s = lax.axis_index("s")
    wid = c * 16 + s
    for k in range(16):
        zbuf[pl.ds(k * 16, 16)] = jnp.zeros((16,), F32)
    for k in range(8):
        ones[pl.ds(k * 16, 16)] = jnp.full((16,), 1.0, F32)
    pltpu.sync_copy(x_hbm, xv)

    def zero_body(t, carry):
        pltpu.sync_copy(zbuf, acc.at[pl.ds(s * ZPT + t * 256, 256)])
        return carry
    lax.fori_loop(0, ZPT // 256, zero_body, 0)
    plsc.subcore_barrier()

    def chunk_body(g, carry):
        row0 = wid * RPT + g * SUBS
        pltpu.sync_copy(src_hbm.at[pl.ds(row0, SUBS)], srcb)
        pltpu.sync_copy(dst_hbm.at[pl.ds(row0, SUBS)], dstb)

        def sub_body(r, carry2):
            for k in range(8):
                sl = pl.ds(k * 16, 16)
                s16 = srcb[r, sl]
                xvals[sl] = plsc.load_gather(xv, [s16])
                d16 = dstb[r, sl] * 2
                idx1[sl] = d16
                idx2[sl] = d16 + 1
            pltpu.sync_copy(ones, acc.at[idx1], add=True)
            pltpu.sync_copy(xvals, acc.at[idx2], add=True)
            return carry2
        lax.fori_loop(0, SUBS, sub_body, 0)
        return carry
    lax.fori_loop(0, CHUNKS, chunk_body, 0)
    plsc.subcore_barrier()

    def dump_body(t, carry):
        off = s * ZPT + t * 256
        pltpu.sync_copy(acc.at[pl.ds(off, 256)], out_hbm.at[c, pl.ds(off, 256)])
        return carry
    lax.fori_loop(0, ZPT // 256, dump_body, 0)


@functools.partial(
    pl.kernel,
    out_type=jax.ShapeDtypeStruct((2, N_PAD, H), F32),
    mesh=_mesh,
    compiler_params=pltpu.CompilerParams(needs_layout_passes=False, use_tc_tiling_on_sc=False),
    scratch_types=[
        pltpu.VMEM((SUBS, LSUB), jnp.int32),
        pltpu.VMEM((SUBS, LSUB), jnp.int32),
        pltpu.VMEM((LSUB, H), F32),
        pltpu.VMEM((LSUB, H), F32),
        pltpu.VMEM_SHARED((N_PAD, H), F32),
    ],
)
def _sc_seg16(h_hbm, src_hbm, dst_hbm, out_hbm, srcb, dstb, rows, zbuf, acc):
    c = lax.axis_index("c")
    s = lax.axis_index("s")
    wid = c * 16 + s
    zv = jnp.zeros((16,), F32)
    for k in range(LSUB):
        zbuf[k, :] = zv

    def zero_body(t, carry):
        pltpu.sync_copy(zbuf, acc.at[pl.ds(s * ZROWS + t * LSUB, LSUB)])
        return carry
    lax.fori_loop(0, ZROWS // LSUB, zero_body, 0)
    plsc.subcore_barrier()

    def chunk_body(g, carry):
        row0 = wid * RPT + g * SUBS
        pltpu.sync_copy(src_hbm.at[pl.ds(row0, SUBS)], srcb)
        pltpu.sync_copy(dst_hbm.at[pl.ds(row0, SUBS)], dstb)

        def sub_body(r, carry2):
            pltpu.sync_copy(h_hbm.at[srcb.at[r]], rows)
            pltpu.sync_copy(rows, acc.at[dstb.at[r]], add=True)
            return carry2
        lax.fori_loop(0, SUBS, sub_body, 0)
        return carry
    lax.fori_loop(0, CHUNKS, chunk_body, 0)
    plsc.subcore_barrier()

    def dump_body(t, carry):
        off = s * ZROWS + t * LSUB
        pltpu.sync_copy(acc.at[pl.ds(off, LSUB)], out_hbm.at[c, pl.ds(off, LSUB)])
        return carry
    lax.fori_loop(0, ZROWS // LSUB, dump_body, 0)


@functools.partial(
    pl.kernel,
    out_type=jax.ShapeDtypeStruct((E_PAD,), F32),
    mesh=_mesh,
    compiler_params=pltpu.CompilerParams(needs_layout_passes=False, use_tc_tiling_on_sc=False),
    scratch_types=[
        pltpu.VMEM((SUBS, LSUB), jnp.int32),
        pltpu.VMEM((SUBS, LSUB), jnp.int32),
        pltpu.VMEM((LSUB, DEP), F32),
        pltpu.VMEM((LSUB, DEP), F32),
        pltpu.VMEM((LSUB,), F32),
        pltpu.VMEM((72, 16), F32),
    ],
)
def _sc_edge(a_hbm, b_hbm, wbc_hbm, src_hbm, dst_hbm, out_hbm,
             srcb, dstb, arows, brows, obuf, wv):
    c = lax.axis_index("c")
    s = lax.axis_index("s")
    wid = c * 16 + s
    iota = lax.iota(jnp.int32, 16)
    pltpu.sync_copy(wbc_hbm, wv)
    e16s = [iota + k * 16 for k in range(8)]

    def chunk_body(g, carry):
        row0 = wid * RPT + g * SUBS
        pltpu.sync_copy(src_hbm.at[pl.ds(row0, SUBS)], srcb)
        pltpu.sync_copy(dst_hbm.at[pl.ds(row0, SUBS)], dstb)

        def sub_body(r, carry2):
            pltpu.sync_copy(a_hbm.at[srcb.at[r]], arows)
            pltpu.sync_copy(b_hbm.at[dstb.at[r]], brows)
            cvec = wv[68, :]
            accs = [cvec for _ in range(8)]
            for j in range(DE):
                wj = wv[j, :]
                bj = wv[34 + j, :]
                jidx = jnp.full((16,), j, jnp.int32)
                for k in range(8):
                    av = plsc.load_gather(arows, [e16s[k], jidx])
                    bv = plsc.load_gather(brows, [e16s[k], jidx])
                    v = jnp.maximum(av + bv + bj, 0.0)
                    accs[k] = accs[k] + v * wj
            for k in range(8):
                obuf[pl.ds(k * 16, 16)] = accs[k]
            ebase = (wid * RPT + g * SUBS + r) * LSUB
            pltpu.sync_copy(obuf, out_hbm.at[pl.ds(ebase, LSUB)])
            return carry2
        lax.fori_loop(0, SUBS, sub_body, 0)
        return carry
    lax.fori_loop(0, CHUNKS, chunk_body, 0)


def _tc_layer1(part, x_pad, Wl, bl, Wr):
    def body(p_ref, x_ref, wl_ref, bl_ref, wr_ref, h_ref, d_ref):
        p = p_ref[...]
        cnt = p[0, :, 0] + p[1, :, 0]
        agg = p[0, :, 1] + p[1, :, 1]
        dcl = jnp.maximum(cnt, 1.0)
        mean = (agg / dcl)[:, None]
        h = mean * wl_ref[...] + bl_ref[...] + x_ref[...] * wr_ref[...]
        h_ref[...] = jnp.maximum(h, 0.0)
        d_ref[...] = dcl[:, None]

    return pl.pallas_call(
        body,
        grid=(GRID_N,),
        in_specs=[
            pl.BlockSpec((2, BN, 2), lambda i: (0, i, 0)),
            pl.BlockSpec((BN, 1), lambda i: (i, 0)),
            pl.BlockSpec((1, H), lambda i: (0, 0)),
            pl.BlockSpec((1, H), lambda i: (0, 0)),
            pl.BlockSpec((1, H), lambda i: (0, 0)),
        ],
        out_specs=[
            pl.BlockSpec((BN, H), lambda i: (i, 0)),
            pl.BlockSpec((BN, 1), lambda i: (i, 0)),
        ],
        out_shape=[
            jax.ShapeDtypeStruct((N_PAD, H), F32),
            jax.ShapeDtypeStruct((N_PAD, 1), F32),
        ],
    )(part, x_pad, Wl, bl, Wr)


def _tc_layer(part, deg, h_prev, Wl, bl, Wr):
    def body(p_ref, d_ref, h_ref, wl_ref, bl_ref, wr_ref, o_ref):
        p = p_ref[...]
        mean = (p[0] + p[1]) / d_ref[...]
        o = (jnp.dot(mean, wl_ref[...], preferred_element_type=F32)
             + bl_ref[...]
             + jnp.dot(h_ref[...], wr_ref[...], preferred_element_type=F32))
        o_ref[...] = jnp.maximum(o, 0.0)

    return pl.pallas_call(
        body,
        grid=(GRID_N,),
        in_specs=[
            pl.BlockSpec((2, BN, H), lambda i: (0, i, 0)),
            pl.BlockSpec((BN, 1), lambda i: (i, 0)),
            pl.BlockSpec((BN, H), lambda i: (i, 0)),
            pl.BlockSpec((H, H), lambda i: (0, 0)),
            pl.BlockSpec((1, H), lambda i: (0, 0)),
            pl.BlockSpec((H, H), lambda i: (0, 0)),
        ],
        out_specs=pl.BlockSpec((BN, H), lambda i: (i, 0)),
        out_shape=jax.ShapeDtypeStruct((N_PAD, H), F32),
    )(part, deg, h_prev, Wl, bl, Wr)


def _tc_final(part, deg, h_prev, x_pad, Wl, bl, Wr, nW, nb, ew0, ew1, ews, ewd):
    def body(p_ref, d_ref, h_ref, x_ref, wl_ref, bl_ref, wr_ref,
             nw_ref, nb_ref, e0_ref, e1_ref, es_ref, ed_ref,
             no_ref, a_ref, b_ref):
        p = p_ref[...]
        mean = (p[0] + p[1]) / d_ref[...]
        h3 = (jnp.dot(mean, wl_ref[...], preferred_element_type=F32)
              + bl_ref[...]
              + jnp.dot(h_ref[...], wr_ref[...], preferred_element_type=F32))
        h3 = jnp.maximum(h3, 0.0)
        no_ref[...] = jnp.dot(h3, nw_ref[...], preferred_element_type=F32) + nb_ref[...]
        xb = x_ref[...]
        a_ref[...] = xb * e0_ref[...] + jnp.dot(h3, es_ref[...], preferred_element_type=F32)
        b_ref[...] = xb * e1_ref[...] + jnp.dot(h3, ed_ref[...], preferred_element_type=F32)

    return pl.pallas_call(
        body,
        grid=(GRID_N,),
        in_specs=[
            pl.BlockSpec((2, BN, H), lambda i: (0, i, 0)),
            pl.BlockSpec((BN, 1), lambda i: (i, 0)),
            pl.BlockSpec((BN, H), lambda i: (i, 0)),
            pl.BlockSpec((BN, 1), lambda i: (i, 0)),
            pl.BlockSpec((H, H), lambda i: (0, 0)),
            pl.BlockSpec((1, H), lambda i: (0, 0)),
            pl.BlockSpec((H, H), lambda i: (0, 0)),
            pl.BlockSpec((H, 1), lambda i: (0, 0)),
            pl.BlockSpec((1, 1), lambda i: (0, 0)),
            pl.BlockSpec((1, DEP), lambda i: (0, 0)),
            pl.BlockSpec((1, DEP), lambda i: (0, 0)),
            pl.BlockSpec((H, DEP), lambda i: (0, 0)),
            pl.BlockSpec((H, DEP), lambda i: (0, 0)),
        ],
        out_specs=[
            pl.BlockSpec((BN, 1), lambda i: (i, 0)),
            pl.BlockSpec((BN, DEP), lambda i: (i, 0)),
            pl.BlockSpec((BN, DEP), lambda i: (i, 0)),
        ],
        out_shape=[
            jax.ShapeDtypeStruct((N_PAD, 1), F32),
            jax.ShapeDtypeStruct((N_PAD, DEP), F32),
            jax.ShapeDtypeStruct((N_PAD, DEP), F32),
        ],
    )(part, deg, h_prev, x_pad, Wl, bl, Wr, nW, nb, ew0, ew1, ews, ewd)


def kernel(x, edge_index,
           conv1_Wl, conv1_bl, conv1_Wr,
           conv2_Wl, conv2_bl, conv2_Wr,
           conv3_Wl, conv3_bl, conv3_Wr,
           edge_W, edge_b, node_W, node_b, ecls_W, ecls_b):
    src = edge_index[0]
    dst = edge_index[1]
    pad_e = E_PAD - E
    src2 = jnp.concatenate([src, jnp.zeros((pad_e,), jnp.int32)]).reshape(ROWS, LSUB)
    dst2 = jnp.concatenate([dst, jnp.full((pad_e,), N, jnp.int32)]).reshape(ROWS, LSUB)
    x_pad = jnp.concatenate([x, jnp.zeros((N_PAD - N, 1), F32)])
    x_flat = x_pad[:, 0]

    part1 = _sc_deg_agg(x_flat, src2, dst2).reshape(2, N_PAD, 2)
    h1, deg = _tc_layer1(part1, x_pad, conv1_Wl, conv1_bl.reshape(1, H), conv1_Wr)
    part2 = _sc_seg16(h1, src2, dst2)
    h2 = _tc_layer(part2, deg, h1, conv2_Wl, conv2_bl.reshape(1, H), conv2_Wr)
    part3 = _sc_seg16(h2, src2, dst2)
    ewp = jnp.pad(edge_W, ((0, 0), (0, DEP - DE)))
    node_full, atab, btab = _tc_final(
        part3, deg, h2, x_pad,
        conv3_Wl, conv3_bl.reshape(1, H), conv3_Wr,
        node_W, node_b.reshape(1, 1),
        ewp[0].reshape(1, DEP), ewp[1].reshape(1, DEP),
        ewp[2:2 + H], ewp[2 + H:2 + 2 * H])

    w = ecls_W[:, 0]
    wbc = jnp.concatenate([
        jnp.broadcast_to(w[:, None], (DE, 16)),
        jnp.broadcast_to(edge_b[:, None], (DE, 16)),
        jnp.broadcast_to(ecls_b.reshape(1, 1), (1, 16)),
        jnp.zeros((3, 16), F32),
    ])
    eout = _sc_edge(atab, btab, wbc, src2, dst2)

    return (node_full[:N], eout[:E][:, None])


# speedup vs baseline: 29.4952x; 1.4616x over previous
import functools

import jax
import jax.numpy as jnp
from jax import lax
from jax.experimental import pallas as pl
from jax.experimental.pallas import tpu as pltpu
from jax.experimental.pallas import tpu_sc as plsc

N = 100000
E = 3200000
H = 16
DE = 2 * H + 2
DEP = 48

NTILES = 32
LSUB = 128
SUBS = 16
CHUNKS = 49
RPT = SUBS * CHUNKS
ROWS = NTILES * RPT
E_PAD = ROWS * LSUB
N_PAD = 100352
ZROWS = N_PAD // 16
BN = 2048
GRID_N = N_PAD // BN

_mesh = plsc.VectorSubcoreMesh(core_axis_name="c", subcore_axis_name="s")
F32 = jnp.float32


ZTOT = 2 * N_PAD
ZPT = ZTOT // 16


@functools.partial(
    pl.kernel,
    out_type=jax.ShapeDtypeStruct((2, ZTOT), F32),
    mesh=_mesh,
    compiler_params=pltpu.CompilerParams(needs_layout_passes=False, use_tc_tiling_on_sc=False),
    scratch_types=[
        pltpu.VMEM((N_PAD,), F32),
        pltpu.VMEM((SUBS, LSUB), jnp.int32),
        pltpu.VMEM((SUBS, LSUB), jnp.int32),
        pltpu.VMEM((LSUB,), jnp.int32),
        pltpu.VMEM((LSUB,), jnp.int32),
        pltpu.VMEM((LSUB,), F32),
        pltpu.VMEM((LSUB,), F32),
        pltpu.VMEM((256,), F32),
        pltpu.VMEM_SHARED((ZTOT,), F32),
    ],
)
def _sc_deg_agg(x_hbm, src_hbm, dst_hbm, out_hbm,
                xv, srcb, dstb, idx1, idx2, ones, xvals, zbuf, acc):
    c = lax.axis_index("c")
    s = lax.axis_index("s")
    wid = c * 16 + s
    for k in range(16):
        zbuf[pl.ds(k * 16, 16)] = jnp.zeros((16,), F32)
    for k in range(8):
        ones[pl.ds(k * 16, 16)] = jnp.full((16,), 1.0, F32)
    pltpu.sync_copy(x_hbm, xv)

    def zero_body(t, carry):
        pltpu.sync_copy(zbuf, acc.at[pl.ds(s * ZPT + t * 256, 256)])
        return carry
    lax.fori_loop(0, ZPT // 256, zero_body, 0)
    plsc.subcore_barrier()

    def chunk_body(g, carry):
        row0 = wid * RPT + g * SUBS
        pltpu.sync_copy(src_hbm.at[pl.ds(row0, SUBS)], srcb)
        pltpu.sync_copy(dst_hbm.at[pl.ds(row0, SUBS)], dstb)

        def sub_body(r, carry2):
            for k in range(8):
                sl = pl.ds(k * 16, 16)
                s16 = srcb[r, sl]
                xvals[sl] = plsc.load_gather(xv, [s16])
                d16 = dstb[r, sl] * 2
                idx1[sl] = d16
                idx2[sl] = d16 + 1
            pltpu.sync_copy(ones, acc.at[idx1], add=True)
            pltpu.sync_copy(xvals, acc.at[idx2], add=True)
            return carry2
        lax.fori_loop(0, SUBS, sub_body, 0)
        return carry
    lax.fori_loop(0, CHUNKS, chunk_body, 0)
    plsc.subcore_barrier()

    def dump_body(t, carry):
        off = s * ZPT + t * 256
        pltpu.sync_copy(acc.at[pl.ds(off, 256)], out_hbm.at[c, pl.ds(off, 256)])
        return carry
    lax.fori_loop(0, ZPT // 256, dump_body, 0)


@functools.partial(
    pl.kernel,
    out_type=jax.ShapeDtypeStruct((2, N_PAD, H), F32),
    mesh=_mesh,
    compiler_params=pltpu.CompilerParams(needs_layout_passes=False, use_tc_tiling_on_sc=False),
    scratch_types=[
        pltpu.VMEM((SUBS, LSUB), jnp.int32),
        pltpu.VMEM((SUBS, LSUB), jnp.int32),
        pltpu.VMEM((LSUB, H), F32),
        pltpu.VMEM((LSUB, H), F32),
        pltpu.VMEM((LSUB, H), F32),
        pltpu.VMEM((LSUB, H), F32),
        pltpu.VMEM((LSUB, H), F32),
        pltpu.SemaphoreType.DMA,
        pltpu.SemaphoreType.DMA,
        pltpu.SemaphoreType.DMA,
        pltpu.SemaphoreType.DMA,
        pltpu.VMEM_SHARED((N_PAD, H), F32),
    ],
)
def _sc_seg16(h_hbm, src_hbm, dst_hbm, out_hbm, srcb, dstb,
              rb0, rb1, rb2, rb3, zbuf, sm0, sm1, sm2, sm3, acc):
    rbufs = (rb0, rb1, rb2, rb3)
    sems = (sm0, sm1, sm2, sm3)
    c = lax.axis_index("c")
    s = lax.axis_index("s")
    wid = c * 16 + s
    zv = jnp.zeros((16,), F32)
    for k in range(LSUB):
        zbuf[k, :] = zv

    def zero_body(t, carry):
        pltpu.sync_copy(zbuf, acc.at[pl.ds(s * ZROWS + t * LSUB, LSUB)])
        return carry
    lax.fori_loop(0, ZROWS // LSUB, zero_body, 0)
    plsc.subcore_barrier()

    def chunk_body(g, carry):
        row0 = wid * RPT + g * SUBS
        pltpu.sync_copy(src_hbm.at[pl.ds(row0, SUBS)], srcb)
        pltpu.sync_copy(dst_hbm.at[pl.ds(row0, SUBS)], dstb)

        for p in range(3):
            pltpu.async_copy(h_hbm.at[srcb.at[p]], rbufs[p], sems[p])

        def quad_body(q, carry2):
            for p in range(4):
                r = 4 * q + p
                pltpu.make_async_copy(h_hbm.at[srcb.at[r]], rbufs[p], sems[p]).wait()

                @pl.when(r + 3 < SUBS)
                def _():
                    pltpu.async_copy(h_hbm.at[srcb.at[r + 3]],
                                     rbufs[(p + 3) % 4], sems[(p + 3) % 4])
                pltpu.sync_copy(rbufs[p], acc.at[dstb.at[r]], add=True)
            return carry2
        lax.fori_loop(0, SUBS // 4, quad_body, 0)
        return carry
    lax.fori_loop(0, CHUNKS, chunk_body, 0)
    plsc.subcore_barrier()

    def dump_body(t, carry):
        off = s * ZROWS + t * LSUB
        pltpu.sync_copy(acc.at[pl.ds(off, LSUB)], out_hbm.at[c, pl.ds(off, LSUB)])
        return carry
    lax.fori_loop(0, ZROWS // LSUB, dump_body, 0)


@functools.partial(
    pl.kernel,
    out_type=jax.ShapeDtypeStruct((E_PAD,), F32),
    mesh=_mesh,
    compiler_params=pltpu.CompilerParams(needs_layout_passes=False, use_tc_tiling_on_sc=False),
    scratch_types=[
        pltpu.VMEM((SUBS, LSUB), jnp.int32),
        pltpu.VMEM((SUBS, LSUB), jnp.int32),
        pltpu.VMEM((LSUB, DEP), F32),
        pltpu.VMEM((LSUB, DEP), F32),
        pltpu.VMEM((LSUB, DEP), F32),
        pltpu.VMEM((LSUB, DEP), F32),
        pltpu.VMEM((SUBS * LSUB,), F32),
        pltpu.VMEM((72, 16), F32),
        pltpu.SemaphoreType.DMA,
        pltpu.SemaphoreType.DMA,
    ],
)
def _sc_edge(a_hbm, b_hbm, wbc_hbm, src_hbm, dst_hbm, out_hbm,
             srcb, dstb, ar0, ar1, br0, br1, obuf, wv, sm0, sm1):
    c = lax.axis_index("c")
    s = lax.axis_index("s")
    wid = c * 16 + s
    iota = lax.iota(jnp.int32, 16)
    pltpu.sync_copy(wbc_hbm, wv)
    e16s = [iota + k * 16 for k in range(8)]
    abufs = (ar0, ar1)
    bbufs = (br0, br1)
    sems = (sm0, sm1)

    def compute(arows, brows, r):
        cvec = wv[68, :]
        accs = [cvec for _ in range(8)]
        for j in range(DE):
            wj = wv[j, :]
            bj = wv[34 + j, :]
            jidx = jnp.full((16,), j, jnp.int32)
            for k in range(8):
                av = plsc.load_gather(arows, [e16s[k], jidx])
                bv = plsc.load_gather(brows, [e16s[k], jidx])
                v = jnp.maximum(av + bv + bj, 0.0)
                accs[k] = accs[k] + v * wj
        for k in range(8):
            obuf[pl.ds(r * LSUB + k * 16, 16)] = accs[k]

    def chunk_body(g, carry):
        row0 = wid * RPT + g * SUBS
        pltpu.sync_copy(src_hbm.at[pl.ds(row0, SUBS)], srcb)
        pltpu.sync_copy(dst_hbm.at[pl.ds(row0, SUBS)], dstb)
        pltpu.async_copy(a_hbm.at[srcb.at[0]], ar0, sm0)
        pltpu.async_copy(b_hbm.at[dstb.at[0]], br0, sm0)

        def pair_body(q, carry2):
            for p in range(2):
                r = 2 * q + p
                pltpu.make_async_copy(a_hbm.at[srcb.at[r]], abufs[p], sems[p]).wait()
                pltpu.make_async_copy(b_hbm.at[dstb.at[r]], bbufs[p], sems[p]).wait()

                @pl.when(r + 1 < SUBS)
                def _():
                    pltpu.async_copy(a_hbm.at[srcb.at[r + 1]], abufs[1 - p], sems[1 - p])
                    pltpu.async_copy(b_hbm.at[dstb.at[r + 1]], bbufs[1 - p], sems[1 - p])
                compute(abufs[p], bbufs[p], r)
            return carry2
        lax.fori_loop(0, SUBS // 2, pair_body, 0)
        ebase = (wid * RPT + g * SUBS) * LSUB
        pltpu.sync_copy(obuf, out_hbm.at[pl.ds(ebase, SUBS * LSUB)])
        return carry
    lax.fori_loop(0, CHUNKS, chunk_body, 0)


def _tc_layer1(part, x_pad, Wl, bl, Wr):
    def body(p_ref, x_ref, wl_ref, bl_ref, wr_ref, h_ref, d_ref):
        p = p_ref[...]
        cnt = p[0, :, 0] + p[1, :, 0]
        agg = p[0, :, 1] + p[1, :, 1]
        dcl = jnp.maximum(cnt, 1.0)
        mean = (agg / dcl)[:, None]
        h = mean * wl_ref[...] + bl_ref[...] + x_ref[...] * wr_ref[...]
        h_ref[...] = jnp.maximum(h, 0.0)
        d_ref[...] = dcl[:, None]

    return pl.pallas_call(
        body,
        grid=(GRID_N,),
        in_specs=[
            pl.BlockSpec((2, BN, 2), lambda i: (0, i, 0)),
            pl.BlockSpec((BN, 1), lambda i: (i, 0)),
            pl.BlockSpec((1, H), lambda i: (0, 0)),
            pl.BlockSpec((1, H), lambda i: (0, 0)),
            pl.BlockSpec((1, H), lambda i: (0, 0)),
        ],
        out_specs=[
            pl.BlockSpec((BN, H), lambda i: (i, 0)),
            pl.BlockSpec((BN, 1), lambda i: (i, 0)),
        ],
        out_shape=[
            jax.ShapeDtypeStruct((N_PAD, H), F32),
            jax.ShapeDtypeStruct((N_PAD, 1), F32),
        ],
    )(part, x_pad, Wl, bl, Wr)


def _tc_layer(part, deg, h_prev, Wl, bl, Wr):
    def body(p_ref, d_ref, h_ref, wl_ref, bl_ref, wr_ref, o_ref):
        p = p_ref[...]
        mean = (p[0] + p[1]) / d_ref[...]
        o = (jnp.dot(mean, wl_ref[...], preferred_element_type=F32)
             + bl_ref[...]
             + jnp.dot(h_ref[...], wr_ref[...], preferred_element_type=F32))
        o_ref[...] = jnp.maximum(o, 0.0)

    return pl.pallas_call(
        body,
        grid=(GRID_N,),
        in_specs=[
            pl.BlockSpec((2, BN, H), lambda i: (0, i, 0)),
            pl.BlockSpec((BN, 1), lambda i: (i, 0)),
            pl.BlockSpec((BN, H), lambda i: (i, 0)),
            pl.BlockSpec((H, H), lambda i: (0, 0)),
            pl.BlockSpec((1, H), lambda i: (0, 0)),
            pl.BlockSpec((H, H), lambda i: (0, 0)),
        ],
        out_specs=pl.BlockSpec((BN, H), lambda i: (i, 0)),
        out_shape=jax.ShapeDtypeStruct((N_PAD, H), F32),
    )(part, deg, h_prev, Wl, bl, Wr)


def _tc_final(part, deg, h_prev, x_pad, Wl, bl, Wr, nW, nb, ew0, ew1, ews, ewd):
    def body(p_ref, d_ref, h_ref, x_ref, wl_ref, bl_ref, wr_ref,
             nw_ref, nb_ref, e0_ref, e1_ref, es_ref, ed_ref,
             no_ref, a_ref, b_ref):
        p = p_ref[...]
        mean = (p[0] + p[1]) / d_ref[...]
        h3 = (jnp.dot(mean, wl_ref[...], preferred_element_type=F32)
              + bl_ref[...]
              + jnp.dot(h_ref[...], wr_ref[...], preferred_element_type=F32))
        h3 = jnp.maximum(h3, 0.0)
        no_ref[...] = jnp.dot(h3, nw_ref[...], preferred_element_type=F32) + nb_ref[...]
        xb = x_ref[...]
        a_ref[...] = xb * e0_ref[...] + jnp.dot(h3, es_ref[...], preferred_element_type=F32)
        b_ref[...] = xb * e1_ref[...] + jnp.dot(h3, ed_ref[...], preferred_element_type=F32)

    return pl.pallas_call(
        body,
        grid=(GRID_N,),
        in_specs=[
            pl.BlockSpec((2, BN, H), lambda i: (0, i, 0)),
            pl.BlockSpec((BN, 1), lambda i: (i, 0)),
            pl.BlockSpec((BN, H), lambda i: (i, 0)),
            pl.BlockSpec((BN, 1), lambda i: (i, 0)),
            pl.BlockSpec((H, H), lambda i: (0, 0)),
            pl.BlockSpec((1, H), lambda i: (0, 0)),
            pl.BlockSpec((H, H), lambda i: (0, 0)),
            pl.BlockSpec((H, 1), lambda i: (0, 0)),
            pl.BlockSpec((1, 1), lambda i: (0, 0)),
            pl.BlockSpec((1, DEP), lambda i: (0, 0)),
            pl.BlockSpec((1, DEP), lambda i: (0, 0)),
            pl.BlockSpec((H, DEP), lambda i: (0, 0)),
            pl.BlockSpec((H, DEP), lambda i: (0, 0)),
        ],
        out_specs=[
            pl.BlockSpec((BN, 1), lambda i: (i, 0)),
            pl.BlockSpec((BN, DEP), lambda i: (i, 0)),
            pl.BlockSpec((BN, DEP), lambda i: (i, 0)),
        ],
        out_shape=[
            jax.ShapeDtypeStruct((N_PAD, 1), F32),
            jax.ShapeDtypeStruct((N_PAD, DEP), F32),
            jax.ShapeDtypeStruct((N_PAD, DEP), F32),
        ],
    )(part, deg, h_prev, x_pad, Wl, bl, Wr, nW, nb, ew0, ew1, ews, ewd)


def kernel(x, edge_index,
           conv1_Wl, conv1_bl, conv1_Wr,
           conv2_Wl, conv2_bl, conv2_Wr,
           conv3_Wl, conv3_bl, conv3_Wr,
           edge_W, edge_b, node_W, node_b, ecls_W, ecls_b):
    src = edge_index[0]
    dst = edge_index[1]
    pad_e = E_PAD - E
    src2 = jnp.concatenate([src, jnp.zeros((pad_e,), jnp.int32)]).reshape(ROWS, LSUB)
    dst2 = jnp.concatenate([dst, jnp.full((pad_e,), N, jnp.int32)]).reshape(ROWS, LSUB)
    x_pad = jnp.concatenate([x, jnp.zeros((N_PAD - N, 1), F32)])
    x_flat = x_pad[:, 0]

    part1 = _sc_deg_agg(x_flat, src2, dst2).reshape(2, N_PAD, 2)
    h1, deg = _tc_layer1(part1, x_pad, conv1_Wl, conv1_bl.reshape(1, H), conv1_Wr)
    part2 = _sc_seg16(h1, src2, dst2)
    h2 = _tc_layer(part2, deg, h1, conv2_Wl, conv2_bl.reshape(1, H), conv2_Wr)
    part3 = _sc_seg16(h2, src2, dst2)
    ewp = jnp.pad(edge_W, ((0, 0), (0, DEP - DE)))
    node_full, atab, btab = _tc_final(
        part3, deg, h2, x_pad,
        conv3_Wl, conv3_bl.reshape(1, H), conv3_Wr,
        node_W, node_b.reshape(1, 1),
        ewp[0].reshape(1, DEP), ewp[1].reshape(1, DEP),
        ewp[2:2 + H], ewp[2 + H:2 + 2 * H])

    w = ecls_W[:, 0]
    wbc = jnp.concatenate([
        jnp.broadcast_to(w[:, None], (DE, 16)),
        jnp.broadcast_to(edge_b[:, None], (DE, 16)),
        jnp.broadcast_to(ecls_b.reshape(1, 1), (1, 16)),
        jnp.zeros((3, 16), F32),
    ])
    eout = _sc_edge(atab, btab, wbc, src2, dst2)

    return (node_full[:N], eout[:E][:, None])


# speedup vs baseline: 32.2419x; 1.0931x over previous
import functools

import jax
import jax.numpy as jnp
from jax import lax
from jax.experimental import pallas as pl
from jax.experimental.pallas import tpu as pltpu
from jax.experimental.pallas import tpu_sc as plsc

N = 100000
E = 3200000
H = 16
DE = 2 * H + 2
DEP = 48

NTILES = 32
LSUB = 128
SUBS = 16
CHUNKS = 49
RPT = SUBS * CHUNKS
ROWS = NTILES * RPT
E_PAD = ROWS * LSUB
N_PAD = 100352
ZROWS = N_PAD // 16
BN = 2048
GRID_N = N_PAD // BN

_mesh = plsc.VectorSubcoreMesh(core_axis_name="c", subcore_axis_name="s")
F32 = jnp.float32


ZTOT = 2 * N_PAD
ZPT = ZTOT // 16


@functools.partial(
    pl.kernel,
    out_type=jax.ShapeDtypeStruct((2, ZTOT), F32),
    mesh=_mesh,
    compiler_params=pltpu.CompilerParams(needs_layout_passes=False, use_tc_tiling_on_sc=False),
    scratch_types=[
        pltpu.VMEM((N_PAD,), F32),
        pltpu.VMEM((SUBS, LSUB), jnp.int32),
        pltpu.VMEM((SUBS, LSUB), jnp.int32),
        pltpu.VMEM((LSUB,), jnp.int32),
        pltpu.VMEM((LSUB,), jnp.int32),
        pltpu.VMEM((LSUB,), F32),
        pltpu.VMEM((LSUB,), F32),
        pltpu.VMEM((256,), F32),
        pltpu.VMEM_SHARED((ZTOT,), F32),
    ],
)
def _sc_deg_agg(x_hbm, src_hbm, dst_hbm, out_hbm,
                xv, srcb, dstb, idx1, idx2, ones, xvals, zbuf, acc):
    c = lax.axis_index("c")
    s = lax.axis_index("s")
    wid = c * 16 + s
    for k in range(16):
        zbuf[pl.ds(k * 16, 16)] = jnp.zeros((16,), F32)
    for k in range(8):
        ones[pl.ds(k * 16, 16)] = jnp.full((16,), 1.0, F32)
    pltpu.sync_copy(x_hbm, xv)

    def zero_body(t, carry):
        pltpu.sync_copy(zbuf, acc.at[pl.ds(s * ZPT + t * 256, 256)])
        return carry
    lax.fori_loop(0, ZPT // 256, zero_body, 0)
    plsc.subcore_barrier()

    def chunk_body(g, carry):
        row0 = wid * RPT + g * SUBS
        pltpu.sync_copy(src_hbm.at[pl.ds(row0, SUBS)], srcb)
        pltpu.sync_copy(dst_hbm.at[pl.ds(row0, SUBS)], dstb)

        def sub_body(r, carry2):
            for k in range(8):
                sl = pl.ds(k * 16, 16)
                s16 = srcb[r, sl]
                xvals[sl] = plsc.load_gather(xv, [s16])
                d16 = dstb[r, sl] * 2
                idx1[sl] = d16
                idx2[sl] = d16 + 1
            pltpu.sync_copy(ones, acc.at[idx1], add=True)
            pltpu.sync_copy(xvals, acc.at[idx2], add=True)
            return carry2
        lax.fori_loop(0, SUBS, sub_body, 0)
        return carry
    lax.fori_loop(0, CHUNKS, chunk_body, 0)
    plsc.subcore_barrier()

    def dump_body(t, carry):
        off = s * ZPT + t * 256
        pltpu.sync_copy(acc.at[pl.ds(off, 256)], out_hbm.at[c, pl.ds(off, 256)])
        return carry
    lax.fori_loop(0, ZPT // 256, dump_body, 0)


@functools.partial(
    pl.kernel,
    out_type=jax.ShapeDtypeStruct((2, N_PAD, H), F32),
    mesh=_mesh,
    compiler_params=pltpu.CompilerParams(needs_layout_passes=False, use_tc_tiling_on_sc=False),
    scratch_types=[
        pltpu.VMEM((SUBS, LSUB), jnp.int32),
        pltpu.VMEM((SUBS, LSUB), jnp.int32),
        pltpu.VMEM((LSUB, H), F32),
        pltpu.VMEM((LSUB, H), F32),
        pltpu.VMEM((LSUB, H), F32),
        pltpu.VMEM((LSUB, H), F32),
        pltpu.VMEM((LSUB, H), F32),
        pltpu.SemaphoreType.DMA,
        pltpu.SemaphoreType.DMA,
        pltpu.SemaphoreType.DMA,
        pltpu.SemaphoreType.DMA,
        pltpu.VMEM_SHARED((N_PAD, H), F32),
    ],
)
def _sc_seg16(h_hbm, src_hbm, dst_hbm, out_hbm, srcb, dstb,
              rb0, rb1, rb2, rb3, zbuf, sm0, sm1, sm2, sm3, acc):
    rbufs = (rb0, rb1, rb2, rb3)
    sems = (sm0, sm1, sm2, sm3)
    c = lax.axis_index("c")
    s = lax.axis_index("s")
    wid = c * 16 + s
    zv = jnp.zeros((16,), F32)
    for k in range(LSUB):
        zbuf[k, :] = zv

    def zero_body(t, carry):
        pltpu.sync_copy(zbuf, acc.at[pl.ds(s * ZROWS + t * LSUB, LSUB)])
        return carry
    lax.fori_loop(0, ZROWS // LSUB, zero_body, 0)
    plsc.subcore_barrier()

    def chunk_body(g, carry):
        row0 = wid * RPT + g * SUBS
        pltpu.sync_copy(src_hbm.at[pl.ds(row0, SUBS)], srcb)
        pltpu.sync_copy(dst_hbm.at[pl.ds(row0, SUBS)], dstb)

        for p in range(3):
            pltpu.async_copy(h_hbm.at[srcb.at[p]], rbufs[p], sems[p])

        def quad_body(q, carry2):
            for p in range(4):
                r = 4 * q + p
                pltpu.make_async_copy(h_hbm.at[srcb.at[r]], rbufs[p], sems[p]).wait()

                @pl.when(r + 3 < SUBS)
                def _():
                    pltpu.async_copy(h_hbm.at[srcb.at[r + 3]],
                                     rbufs[(p + 3) % 4], sems[(p + 3) % 4])
                pltpu.sync_copy(rbufs[p], acc.at[dstb.at[r]], add=True)
            return carry2
        lax.fori_loop(0, SUBS // 4, quad_body, 0)
        return carry
    lax.fori_loop(0, CHUNKS, chunk_body, 0)
    plsc.subcore_barrier()

    def dump_body(t, carry):
        off = s * ZROWS + t * LSUB
        pltpu.sync_copy(acc.at[pl.ds(off, LSUB)], out_hbm.at[c, pl.ds(off, LSUB)])
        return carry
    lax.fori_loop(0, ZROWS // LSUB, dump_body, 0)


@functools.partial(
    pl.kernel,
    out_type=jax.ShapeDtypeStruct((E_PAD,), F32),
    mesh=_mesh,
    compiler_params=pltpu.CompilerParams(needs_layout_passes=False, use_tc_tiling_on_sc=False),
    scratch_types=[
        pltpu.VMEM((SUBS, LSUB), jnp.int32),
        pltpu.VMEM((SUBS, LSUB), jnp.int32),
        pltpu.VMEM((LSUB, DEP), F32),
        pltpu.VMEM((LSUB, DEP), F32),
        pltpu.VMEM((LSUB, DEP), F32),
        pltpu.VMEM((LSUB, DEP), F32),
        pltpu.VMEM((SUBS * LSUB,), F32),
        pltpu.VMEM((96,), F32),
        pltpu.SemaphoreType.DMA,
        pltpu.SemaphoreType.DMA,
    ],
)
def _sc_edge(a_hbm, b_hbm, wbc_hbm, src_hbm, dst_hbm, out_hbm,
             srcb, dstb, ar0, ar1, br0, br1, obuf, wv, sm0, sm1):
    c = lax.axis_index("c")
    s = lax.axis_index("s")
    wid = c * 16 + s
    iota = lax.iota(jnp.int32, 16)
    pltpu.sync_copy(wbc_hbm, wv)
    e16s = [iota + k * 16 for k in range(8)]
    abufs = (ar0, ar1)
    bbufs = (br0, br1)
    sems = (sm0, sm1)

    def compute(arows, brows, r):
        cvec = wv[pl.ds(80, 16)]
        accs = [cvec for _ in range(8)]
        jvec = iota
        dec = jnp.full((16,), DE, jnp.int32)
        for j in range(DE):
            wcol = plsc.load_gather(wv, [jvec])
            bcol = plsc.load_gather(wv, [jvec + 40])
            for k in range(8):
                av = plsc.load_gather(arows, [e16s[k], jvec])
                bv = plsc.load_gather(brows, [e16s[k], jvec])
                v = jnp.maximum(av + bv + bcol, 0.0)
                accs[k] = accs[k] + v * wcol
            jnext = jvec + 1
            jvec = jnp.where(jnext >= dec, jnext - dec, jnext)
        for k in range(8):
            obuf[pl.ds(r * LSUB + k * 16, 16)] = accs[k]

    def chunk_body(g, carry):
        row0 = wid * RPT + g * SUBS
        pltpu.sync_copy(src_hbm.at[pl.ds(row0, SUBS)], srcb)
        pltpu.sync_copy(dst_hbm.at[pl.ds(row0, SUBS)], dstb)
        pltpu.async_copy(a_hbm.at[srcb.at[0]], ar0, sm0)
        pltpu.async_copy(b_hbm.at[dstb.at[0]], br0, sm0)

        def pair_body(q, carry2):
            for p in range(2):
                r = 2 * q + p
                pltpu.make_async_copy(a_hbm.at[srcb.at[r]], abufs[p], sems[p]).wait()
                pltpu.make_async_copy(b_hbm.at[dstb.at[r]], bbufs[p], sems[p]).wait()

                @pl.when(r + 1 < SUBS)
                def _():
                    pltpu.async_copy(a_hbm.at[srcb.at[r + 1]], abufs[1 - p], sems[1 - p])
                    pltpu.async_copy(b_hbm.at[dstb.at[r + 1]], bbufs[1 - p], sems[1 - p])
                compute(abufs[p], bbufs[p], r)
            return carry2
        lax.fori_loop(0, SUBS // 2, pair_body, 0)
        ebase = (wid * RPT + g * SUBS) * LSUB
        pltpu.sync_copy(obuf, out_hbm.at[pl.ds(ebase, SUBS * LSUB)])
        return carry
    lax.fori_loop(0, CHUNKS, chunk_body, 0)


def _tc_layer1(part, x_pad, Wl, bl, Wr):
    def body(p_ref, x_ref, wl_ref, bl_ref, wr_ref, h_ref, d_ref):
        p = p_ref[...]
        cnt = p[0, :, 0] + p[1, :, 0]
        agg = p[0, :, 1] + p[1, :, 1]
        dcl = jnp.maximum(cnt, 1.0)
        mean = (agg / dcl)[:, None]
        h = mean * wl_ref[...] + bl_ref[...] + x_ref[...] * wr_ref[...]
        h_ref[...] = jnp.maximum(h, 0.0)
        d_ref[...] = dcl[:, None]

    return pl.pallas_call(
        body,
        grid=(GRID_N,),
        in_specs=[
            pl.BlockSpec((2, BN, 2), lambda i: (0, i, 0)),
            pl.BlockSpec((BN, 1), lambda i: (i, 0)),
            pl.BlockSpec((1, H), lambda i: (0, 0)),
            pl.BlockSpec((1, H), lambda i: (0, 0)),
            pl.BlockSpec((1, H), lambda i: (0, 0)),
        ],
        out_specs=[
            pl.BlockSpec((BN, H), lambda i: (i, 0)),
            pl.BlockSpec((BN, 1), lambda i: (i, 0)),
        ],
        out_shape=[
            jax.ShapeDtypeStruct((N_PAD, H), F32),
            jax.ShapeDtypeStruct((N_PAD, 1), F32),
        ],
    )(part, x_pad, Wl, bl, Wr)


def _tc_layer(part, deg, h_prev, Wl, bl, Wr):
    def body(p_ref, d_ref, h_ref, wl_ref, bl_ref, wr_ref, o_ref):
        p = p_ref[...]
        mean = (p[0] + p[1]) / d_ref[...]
        o = (jnp.dot(mean, wl_ref[...], preferred_element_type=F32)
             + bl_ref[...]
             + jnp.dot(h_ref[...], wr_ref[...], preferred_element_type=F32))
        o_ref[...] = jnp.maximum(o, 0.0)

    return pl.pallas_call(
        body,
        grid=(GRID_N,),
        in_specs=[
            pl.BlockSpec((2, BN, H), lambda i: (0, i, 0)),
            pl.BlockSpec((BN, 1), lambda i: (i, 0)),
            pl.BlockSpec((BN, H), lambda i: (i, 0)),
            pl.BlockSpec((H, H), lambda i: (0, 0)),
            pl.BlockSpec((1, H), lambda i: (0, 0)),
            pl.BlockSpec((H, H), lambda i: (0, 0)),
        ],
        out_specs=pl.BlockSpec((BN, H), lambda i: (i, 0)),
        out_shape=jax.ShapeDtypeStruct((N_PAD, H), F32),
    )(part, deg, h_prev, Wl, bl, Wr)


def _tc_final(part, deg, h_prev, x_pad, Wl, bl, Wr, nW, nb, ew0, ew1, ews, ewd):
    def body(p_ref, d_ref, h_ref, x_ref, wl_ref, bl_ref, wr_ref,
             nw_ref, nb_ref, e0_ref, e1_ref, es_ref, ed_ref,
             no_ref, a_ref, b_ref):
        p = p_ref[...]
        mean = (p[0] + p[1]) / d_ref[...]
        h3 = (jnp.dot(mean, wl_ref[...], preferred_element_type=F32)
              + bl_ref[...]
              + jnp.dot(h_ref[...], wr_ref[...], preferred_element_type=F32))
        h3 = jnp.maximum(h3, 0.0)
        no_ref[...] = jnp.dot(h3, nw_ref[...], preferred_element_type=F32) + nb_ref[...]
        xb = x_ref[...]
        a_ref[...] = xb * e0_ref[...] + jnp.dot(h3, es_ref[...], preferred_element_type=F32)
        b_ref[...] = xb * e1_ref[...] + jnp.dot(h3, ed_ref[...], preferred_element_type=F32)

    return pl.pallas_call(
        body,
        grid=(GRID_N,),
        in_specs=[
            pl.BlockSpec((2, BN, H), lambda i: (0, i, 0)),
            pl.BlockSpec((BN, 1), lambda i: (i, 0)),
            pl.BlockSpec((BN, H), lambda i: (i, 0)),
            pl.BlockSpec((BN, 1), lambda i: (i, 0)),
            pl.BlockSpec((H, H), lambda i: (0, 0)),
            pl.BlockSpec((1, H), lambda i: (0, 0)),
            pl.BlockSpec((H, H), lambda i: (0, 0)),
            pl.BlockSpec((H, 1), lambda i: (0, 0)),
            pl.BlockSpec((1, 1), lambda i: (0, 0)),
            pl.BlockSpec((1, DEP), lambda i: (0, 0)),
            pl.BlockSpec((1, DEP), lambda i: (0, 0)),
            pl.BlockSpec((H, DEP), lambda i: (0, 0)),
            pl.BlockSpec((H, DEP), lambda i: (0, 0)),
        ],
        out_specs=[
            pl.BlockSpec((BN, 1), lambda i: (i, 0)),
            pl.BlockSpec((BN, DEP), lambda i: (i, 0)),
            pl.BlockSpec((BN, DEP), lambda i: (i, 0)),
        ],
        out_shape=[
            jax.ShapeDtypeStruct((N_PAD, 1), F32),
            jax.ShapeDtypeStruct((N_PAD, DEP), F32),
            jax.ShapeDtypeStruct((N_PAD, DEP), F32),
        ],
    )(part, deg, h_prev, x_pad, Wl, bl, Wr, nW, nb, ew0, ew1, ews, ewd)


def kernel(x, edge_index,
           conv1_Wl, conv1_bl, conv1_Wr,
           conv2_Wl, conv2_bl, conv2_Wr,
           conv3_Wl, conv3_bl, conv3_Wr,
           edge_W, edge_b, node_W, node_b, ecls_W, ecls_b):
    src = edge_index[0]
    dst = edge_index[1]
    pad_e = E_PAD - E
    src2 = jnp.concatenate([src, jnp.zeros((pad_e,), jnp.int32)]).reshape(ROWS, LSUB)
    dst2 = jnp.concatenate([dst, jnp.full((pad_e,), N, jnp.int32)]).reshape(ROWS, LSUB)
    x_pad = jnp.concatenate([x, jnp.zeros((N_PAD - N, 1), F32)])
    x_flat = x_pad[:, 0]

    part1 = _sc_deg_agg(x_flat, src2, dst2).reshape(2, N_PAD, 2)
    h1, deg = _tc_layer1(part1, x_pad, conv1_Wl, conv1_bl.reshape(1, H), conv1_Wr)
    part2 = _sc_seg16(h1, src2, dst2)
    h2 = _tc_layer(part2, deg, h1, conv2_Wl, conv2_bl.reshape(1, H), conv2_Wr)
    part3 = _sc_seg16(h2, src2, dst2)
    ewp = jnp.pad(edge_W, ((0, 0), (0, DEP - DE)))
    node_full, atab, btab = _tc_final(
        part3, deg, h2, x_pad,
        conv3_Wl, conv3_bl.reshape(1, H), conv3_Wr,
        node_W, node_b.reshape(1, 1),
        ewp[0].reshape(1, DEP), ewp[1].reshape(1, DEP),
        ewp[2:2 + H], ewp[2 + H:2 + 2 * H])

    w = ecls_W[:, 0]
    wbc = jnp.concatenate([
        w, jnp.zeros((6,), F32),
        edge_b, jnp.zeros((6,), F32),
        jnp.broadcast_to(ecls_b, (16,)),
    ])
    eout = _sc_edge(atab, btab, wbc, src2, dst2)

    return (node_full[:N], eout[:E][:, None])


# speedup vs baseline: 38.5970x; 1.1971x over previous
import functools

import jax
import jax.numpy as jnp
from jax import lax
from jax.experimental import pallas as pl
from jax.experimental.pallas import tpu as pltpu
from jax.experimental.pallas import tpu_sc as plsc

N = 100000
E = 3200000
H = 16
DE = 2 * H + 2
DEP = 48

NTILES = 32
LSUB = 128
SUBS = 16
CHUNKS = 49
RPT = SUBS * CHUNKS
ROWS = NTILES * RPT
E_PAD = ROWS * LSUB
N_PAD = 100352
ZROWS = N_PAD // 16
BN = 2048
GRID_N = N_PAD // BN

_mesh = plsc.VectorSubcoreMesh(core_axis_name="c", subcore_axis_name="s")
F32 = jnp.float32


ZTOT = 2 * N_PAD
ZPT = ZTOT // 16


@functools.partial(
    pl.kernel,
    out_type=jax.ShapeDtypeStruct((2, ZTOT), F32),
    mesh=_mesh,
    compiler_params=pltpu.CompilerParams(needs_layout_passes=False, use_tc_tiling_on_sc=False),
    scratch_types=[
        pltpu.VMEM((N_PAD,), F32),
        pltpu.VMEM((SUBS, LSUB), jnp.int32),
        pltpu.VMEM((SUBS, LSUB), jnp.int32),
        pltpu.VMEM((LSUB,), jnp.int32),
        pltpu.VMEM((LSUB,), jnp.int32),
        pltpu.VMEM((LSUB,), F32),
        pltpu.VMEM((LSUB,), F32),
        pltpu.VMEM((256,), F32),
        pltpu.VMEM_SHARED((ZTOT,), F32),
    ],
)
def _sc_deg_agg(x_hbm, src_hbm, dst_hbm, out_hbm,
                xv, srcb, dstb, idx1, idx2, ones, xvals, zbuf, acc):
    c = lax.axis_index("c")
    s = lax.axis_index("s")
    wid = c * 16 + s
    for k in range(16):
        zbuf[pl.ds(k * 16, 16)] = jnp.zeros((16,), F32)
    for k in range(8):
        ones[pl.ds(k * 16, 16)] = jnp.full((16,), 1.0, F32)
    pltpu.sync_copy(x_hbm, xv)

    def zero_body(t, carry):
        pltpu.sync_copy(zbuf, acc.at[pl.ds(s * ZPT + t * 256, 256)])
        return carry
    lax.fori_loop(0, ZPT // 256, zero_body, 0)
    plsc.subcore_barrier()

    def chunk_body(g, carry):
        row0 = wid * RPT + g * SUBS
        pltpu.sync_copy(src_hbm.at[pl.ds(row0, SUBS)], srcb)
        pltpu.sync_copy(dst_hbm.at[pl.ds(row0, SUBS)], dstb)

        def sub_body(r, carry2):
            for k in range(8):
                sl = pl.ds(k * 16, 16)
                s16 = srcb[r, sl]
                xvals[sl] = plsc.load_gather(xv, [s16])
                d16 = dstb[r, sl] * 2
                idx1[sl] = d16
                idx2[sl] = d16 + 1
            pltpu.sync_copy(ones, acc.at[idx1], add=True)
            pltpu.sync_copy(xvals, acc.at[idx2], add=True)
            return carry2
        lax.fori_loop(0, SUBS, sub_body, 0)
        return carry
    lax.fori_loop(0, CHUNKS, chunk_body, 0)
    plsc.subcore_barrier()

    def dump_body(t, carry):
        off = s * ZPT + t * 256
        pltpu.sync_copy(acc.at[pl.ds(off, 256)], out_hbm.at[c, pl.ds(off, 256)])
        return carry
    lax.fori_loop(0, ZPT // 256, dump_body, 0)


@functools.partial(
    pl.kernel,
    out_type=jax.ShapeDtypeStruct((2, N_PAD, H), F32),
    mesh=_mesh,
    compiler_params=pltpu.CompilerParams(needs_layout_passes=False, use_tc_tiling_on_sc=False),
    scratch_types=[
        pltpu.VMEM((SUBS, LSUB), jnp.int32),
        pltpu.VMEM((SUBS, LSUB), jnp.int32),
        pltpu.VMEM((LSUB, H), F32),
        pltpu.VMEM((LSUB, H), F32),
        pltpu.VMEM((LSUB, H), F32),
        pltpu.VMEM((LSUB, H), F32),
        pltpu.VMEM((LSUB, H), F32),
        pltpu.SemaphoreType.DMA,
        pltpu.SemaphoreType.DMA,
        pltpu.SemaphoreType.DMA,
        pltpu.SemaphoreType.DMA,
        pltpu.VMEM_SHARED((N_PAD, H), F32),
    ],
)
def _sc_seg16(h_hbm, src_hbm, dst_hbm, out_hbm, srcb, dstb,
              rb0, rb1, rb2, rb3, zbuf, sm0, sm1, sm2, sm3, acc):
    rbufs = (rb0, rb1, rb2, rb3)
    sems = (sm0, sm1, sm2, sm3)
    c = lax.axis_index("c")
    s = lax.axis_index("s")
    wid = c * 16 + s
    zv = jnp.zeros((16,), F32)
    for k in range(LSUB):
        zbuf[k, :] = zv

    def zero_body(t, carry):
        pltpu.sync_copy(zbuf, acc.at[pl.ds(s * ZROWS + t * LSUB, LSUB)])
        return carry
    lax.fori_loop(0, ZROWS // LSUB, zero_body, 0)
    plsc.subcore_barrier()

    def chunk_body(g, carry):
        row0 = wid * RPT + g * SUBS
        pltpu.sync_copy(src_hbm.at[pl.ds(row0, SUBS)], srcb)
        pltpu.sync_copy(dst_hbm.at[pl.ds(row0, SUBS)], dstb)

        for p in range(3):
            pltpu.async_copy(h_hbm.at[srcb.at[p]], rbufs[p], sems[p])

        def quad_body(q, carry2):
            for p in range(4):
                r = 4 * q + p
                pltpu.make_async_copy(h_hbm.at[srcb.at[r]], rbufs[p], sems[p]).wait()

                @pl.when(r + 3 < SUBS)
                def _():
                    pltpu.async_copy(h_hbm.at[srcb.at[r + 3]],
                                     rbufs[(p + 3) % 4], sems[(p + 3) % 4])
                pltpu.sync_copy(rbufs[p], acc.at[dstb.at[r]], add=True)
            return carry2
        lax.fori_loop(0, SUBS // 4, quad_body, 0)
        return carry
    lax.fori_loop(0, CHUNKS, chunk_body, 0)
    plsc.subcore_barrier()

    def dump_body(t, carry):
        off = s * ZROWS + t * LSUB
        pltpu.sync_copy(acc.at[pl.ds(off, LSUB)], out_hbm.at[c, pl.ds(off, LSUB)])
        return carry
    lax.fori_loop(0, ZROWS // LSUB, dump_body, 0)


DW = DEP // 2
MW = 17
SUBS_D = 56
CHUNKS_D = RPT // SUBS_D


@functools.partial(
    pl.kernel,
    out_type=jax.ShapeDtypeStruct((E_PAD,), F32),
    mesh=_mesh,
    compiler_params=pltpu.CompilerParams(needs_layout_passes=False, use_tc_tiling_on_sc=False),
    scratch_types=[
        pltpu.VMEM((SUBS_D, LSUB), jnp.int32),
        pltpu.VMEM((SUBS_D, LSUB), jnp.int32),
        pltpu.VMEM((LSUB, DW), jnp.int32),
        pltpu.VMEM((LSUB, DW), jnp.int32),
        pltpu.VMEM((LSUB, DW), jnp.int32),
        pltpu.VMEM((LSUB, DW), jnp.int32),
        pltpu.VMEM((SUBS_D * LSUB,), F32),
        pltpu.VMEM((64,), jnp.int32),
        pltpu.SemaphoreType.DMA,
        pltpu.SemaphoreType.DMA,
    ],
)
def _sc_edge(a_hbm, b_hbm, wbc_hbm, src_hbm, dst_hbm, out_hbm,
             srcb, dstb, ar0, ar1, br0, br1, obuf, wv, sm0, sm1):
    c = lax.axis_index("c")
    s = lax.axis_index("s")
    wid = c * 16 + s
    iota = lax.iota(jnp.int32, 16)
    pltpu.sync_copy(wbc_hbm, wv)
    e16s = [iota + k * 16 for k in range(8)]
    abufs = (ar0, ar1)
    bbufs = (br0, br1)
    sems = (sm0, sm1)
    himask = jnp.full((16,), -65536, jnp.int32)

    def unpack(word):
        lo = plsc.bitcast(lax.shift_left(word, 16), F32)
        hi = plsc.bitcast(lax.bitwise_and(word, himask), F32)
        return lo, hi

    def compute(arows, brows, r):
        cvec = plsc.bitcast(wv[pl.ds(48, 16)], F32)
        accs = [cvec for _ in range(8)]
        mvec = iota
        dec = jnp.full((16,), MW, jnp.int32)
        for m0 in range(MW):
            wlo, whi = unpack(plsc.load_gather(wv, [mvec]))
            blo, bhi = unpack(plsc.load_gather(wv, [mvec + 24]))
            for k in range(8):
                alo, ahi = unpack(plsc.load_gather(arows, [e16s[k], mvec]))
                clo, chi = unpack(plsc.load_gather(brows, [e16s[k], mvec]))
                v0 = jnp.maximum(alo + clo + blo, 0.0)
                v1 = jnp.maximum(ahi + chi + bhi, 0.0)
                accs[k] = accs[k] + v0 * wlo + v1 * whi
            mnext = mvec + 1
            mvec = jnp.where(mnext >= dec, mnext - dec, mnext)
        for k in range(8):
            obuf[pl.ds(r * LSUB + k * 16, 16)] = accs[k]

    def chunk_body(g, carry):
        row0 = wid * RPT + g * SUBS_D
        pltpu.sync_copy(src_hbm.at[pl.ds(row0, SUBS_D)], srcb)
        pltpu.sync_copy(dst_hbm.at[pl.ds(row0, SUBS_D)], dstb)
        pltpu.async_copy(a_hbm.at[srcb.at[0]], ar0, sm0)
        pltpu.async_copy(b_hbm.at[dstb.at[0]], br0, sm0)

        def pair_body(q, carry2):
            for p in range(2):
                r = 2 * q + p
                pltpu.make_async_copy(a_hbm.at[srcb.at[r]], abufs[p], sems[p]).wait()
                pltpu.make_async_copy(b_hbm.at[dstb.at[r]], bbufs[p], sems[p]).wait()

                @pl.when(r + 1 < SUBS_D)
                def _():
                    pltpu.async_copy(a_hbm.at[srcb.at[r + 1]], abufs[1 - p], sems[1 - p])
                    pltpu.async_copy(b_hbm.at[dstb.at[r + 1]], bbufs[1 - p], sems[1 - p])
                compute(abufs[p], bbufs[p], r)
            return carry2
        lax.fori_loop(0, SUBS_D // 2, pair_body, 0)
        ebase = (wid * RPT + g * SUBS_D) * LSUB
        pltpu.sync_copy(obuf, out_hbm.at[pl.ds(ebase, SUBS_D * LSUB)])
        return carry
    lax.fori_loop(0, CHUNKS_D, chunk_body, 0)


def _tc_layer1(part, x_pad, Wl, bl, Wr):
    def body(p_ref, x_ref, wl_ref, bl_ref, wr_ref, h_ref, d_ref):
        p = p_ref[...]
        cnt = p[0, :, 0] + p[1, :, 0]
        agg = p[0, :, 1] + p[1, :, 1]
        dcl = jnp.maximum(cnt, 1.0)
        mean = (agg / dcl)[:, None]
        h = mean * wl_ref[...] + bl_ref[...] + x_ref[...] * wr_ref[...]
        h_ref[...] = jnp.maximum(h, 0.0)
        d_ref[...] = dcl[:, None]

    return pl.pallas_call(
        body,
        grid=(GRID_N,),
        in_specs=[
            pl.BlockSpec((2, BN, 2), lambda i: (0, i, 0)),
            pl.BlockSpec((BN, 1), lambda i: (i, 0)),
            pl.BlockSpec((1, H), lambda i: (0, 0)),
            pl.BlockSpec((1, H), lambda i: (0, 0)),
            pl.BlockSpec((1, H), lambda i: (0, 0)),
        ],
        out_specs=[
            pl.BlockSpec((BN, H), lambda i: (i, 0)),
            pl.BlockSpec((BN, 1), lambda i: (i, 0)),
        ],
        out_shape=[
            jax.ShapeDtypeStruct((N_PAD, H), F32),
            jax.ShapeDtypeStruct((N_PAD, 1), F32),
        ],
    )(part, x_pad, Wl, bl, Wr)


def _tc_layer(part, deg, h_prev, Wl, bl, Wr):
    def body(p_ref, d_ref, h_ref, wl_ref, bl_ref, wr_ref, o_ref):
        p = p_ref[...]
        mean = (p[0] + p[1]) / d_ref[...]
        o = (jnp.dot(mean, wl_ref[...], preferred_element_type=F32)
             + bl_ref[...]
             + jnp.dot(h_ref[...], wr_ref[...], preferred_element_type=F32))
        o_ref[...] = jnp.maximum(o, 0.0)

    return pl.pallas_call(
        body,
        grid=(GRID_N,),
        in_specs=[
            pl.BlockSpec((2, BN, H), lambda i: (0, i, 0)),
            pl.BlockSpec((BN, 1), lambda i: (i, 0)),
            pl.BlockSpec((BN, H), lambda i: (i, 0)),
            pl.BlockSpec((H, H), lambda i: (0, 0)),
            pl.BlockSpec((1, H), lambda i: (0, 0)),
            pl.BlockSpec((H, H), lambda i: (0, 0)),
        ],
        out_specs=pl.BlockSpec((BN, H), lambda i: (i, 0)),
        out_shape=jax.ShapeDtypeStruct((N_PAD, H), F32),
    )(part, deg, h_prev, Wl, bl, Wr)


def _tc_final(part, deg, h_prev, x_pad, Wl, bl, Wr, nW, nb, ew0, ew1, ews, ewd):
    def body(p_ref, d_ref, h_ref, x_ref, wl_ref, bl_ref, wr_ref,
             nw_ref, nb_ref, e0_ref, e1_ref, es_ref, ed_ref,
             no_ref, a_ref, b_ref):
        p = p_ref[...]
        mean = (p[0] + p[1]) / d_ref[...]
        h3 = (jnp.dot(mean, wl_ref[...], preferred_element_type=F32)
              + bl_ref[...]
              + jnp.dot(h_ref[...], wr_ref[...], preferred_element_type=F32))
        h3 = jnp.maximum(h3, 0.0)
        no_ref[...] = jnp.dot(h3, nw_ref[...], preferred_element_type=F32) + nb_ref[...]
        xb = x_ref[...]
        a_ref[...] = xb * e0_ref[...] + jnp.dot(h3, es_ref[...], preferred_element_type=F32)
        b_ref[...] = xb * e1_ref[...] + jnp.dot(h3, ed_ref[...], preferred_element_type=F32)

    return pl.pallas_call(
        body,
        grid=(GRID_N,),
        in_specs=[
            pl.BlockSpec((2, BN, H), lambda i: (0, i, 0)),
            pl.BlockSpec((BN, 1), lambda i: (i, 0)),
            pl.BlockSpec((BN, H), lambda i: (i, 0)),
            pl.BlockSpec((BN, 1), lambda i: (i, 0)),
            pl.BlockSpec((H, H), lambda i: (0, 0)),
            pl.BlockSpec((1, H), lambda i: (0, 0)),
            pl.BlockSpec((H, H), lambda i: (0, 0)),
            pl.BlockSpec((H, 1), lambda i: (0, 0)),
            pl.BlockSpec((1, 1), lambda i: (0, 0)),
            pl.BlockSpec((1, DEP), lambda i: (0, 0)),
            pl.BlockSpec((1, DEP), lambda i: (0, 0)),
            pl.BlockSpec((H, DEP), lambda i: (0, 0)),
            pl.BlockSpec((H, DEP), lambda i: (0, 0)),
        ],
        out_specs=[
            pl.BlockSpec((BN, 1), lambda i: (i, 0)),
            pl.BlockSpec((BN, DEP), lambda i: (i, 0)),
            pl.BlockSpec((BN, DEP), lambda i: (i, 0)),
        ],
        out_shape=[
            jax.ShapeDtypeStruct((N_PAD, 1), F32),
            jax.ShapeDtypeStruct((N_PAD, DEP), F32),
            jax.ShapeDtypeStruct((N_PAD, DEP), F32),
        ],
    )(part, deg, h_prev, x_pad, Wl, bl, Wr, nW, nb, ew0, ew1, ews, ewd)


def kernel(x, edge_index,
           conv1_Wl, conv1_bl, conv1_Wr,
           conv2_Wl, conv2_bl, conv2_Wr,
           conv3_Wl, conv3_bl, conv3_Wr,
           edge_W, edge_b, node_W, node_b, ecls_W, ecls_b):
    src = edge_index[0]
    dst = edge_index[1]
    pad_e = E_PAD - E
    src2 = jnp.concatenate([src, jnp.zeros((pad_e,), jnp.int32)]).reshape(ROWS, LSUB)
    dst2 = jnp.concatenate([dst, jnp.full((pad_e,), N, jnp.int32)]).reshape(ROWS, LSUB)
    x_pad = jnp.concatenate([x, jnp.zeros((N_PAD - N, 1), F32)])
    x_flat = x_pad[:, 0]

    part1 = _sc_deg_agg(x_flat, src2, dst2).reshape(2, N_PAD, 2)
    h1, deg = _tc_layer1(part1, x_pad, conv1_Wl, conv1_bl.reshape(1, H), conv1_Wr)
    part2 = _sc_seg16(h1, src2, dst2)
    h2 = _tc_layer(part2, deg, h1, conv2_Wl, conv2_bl.reshape(1, H), conv2_Wr)
    part3 = _sc_seg16(h2, src2, dst2)
    ewp = jnp.pad(edge_W, ((0, 0), (0, DEP - DE)))
    node_full, atab, btab = _tc_final(
        part3, deg, h2, x_pad,
        conv3_Wl, conv3_bl.reshape(1, H), conv3_Wr,
        node_W, node_b.reshape(1, 1),
        ewp[0].reshape(1, DEP), ewp[1].reshape(1, DEP),
        ewp[2:2 + H], ewp[2 + H:2 + 2 * H])

    bf16 = jnp.bfloat16
    atab_p = lax.bitcast_convert_type(
        atab.astype(bf16).reshape(N_PAD, DW, 2), jnp.int32)
    btab_p = lax.bitcast_convert_type(
        btab.astype(bf16).reshape(N_PAD, DW, 2), jnp.int32)
    wpack = lax.bitcast_convert_type(
        jnp.pad(ecls_W[:, 0], (0, DEP - DE)).astype(bf16).reshape(DW, 2), jnp.int32)
    bpack = lax.bitcast_convert_type(
        jnp.pad(edge_b, (0, DEP - DE)).astype(bf16).reshape(DW, 2), jnp.int32)
    cbits = lax.bitcast_convert_type(jnp.broadcast_to(ecls_b, (16,)), jnp.int32)
    wbc = jnp.concatenate([wpack, bpack, cbits])
    eout = _sc_edge(atab_p, btab_p, wbc, src2, dst2)

    return (node_full[:N], eout[:E][:, None])


# speedup vs baseline: 38.7489x; 1.0039x over previous
import functools

import jax
import jax.numpy as jnp
from jax import lax
from jax.experimental import pallas as pl
from jax.experimental.pallas import tpu as pltpu
from jax.experimental.pallas import tpu_sc as plsc

N = 100000
E = 3200000
H = 16
DE = 2 * H + 2
DEP = 48

NTILES = 32
LSUB = 128
SUBS = 16
CHUNKS = 49
RPT = SUBS * CHUNKS
ROWS = NTILES * RPT
E_PAD = ROWS * LSUB
N_PAD = 100352
ZROWS = N_PAD // 16
BN = 6272
GRID_N = N_PAD // BN

_mesh = plsc.VectorSubcoreMesh(core_axis_name="c", subcore_axis_name="s")
F32 = jnp.float32


ZTOT = 2 * N_PAD
ZPT = ZTOT // 16


@functools.partial(
    pl.kernel,
    out_type=jax.ShapeDtypeStruct((2, ZTOT), F32),
    mesh=_mesh,
    compiler_params=pltpu.CompilerParams(needs_layout_passes=False, use_tc_tiling_on_sc=False),
    scratch_types=[
        pltpu.VMEM((N_PAD,), F32),
        pltpu.VMEM((SUBS, LSUB), jnp.int32),
        pltpu.VMEM((SUBS, LSUB), jnp.int32),
        pltpu.VMEM((LSUB,), jnp.int32),
        pltpu.VMEM((LSUB,), jnp.int32),
        pltpu.VMEM((LSUB,), F32),
        pltpu.VMEM((LSUB,), F32),
        pltpu.VMEM((256,), F32),
        pltpu.VMEM_SHARED((ZTOT,), F32),
    ],
)
def _sc_deg_agg(x_hbm, src_hbm, dst_hbm, out_hbm,
                xv, srcb, dstb, idx1, idx2, ones, xvals, zbuf, acc):
    c = lax.axis_index("c")
    s = lax.axis_index("s")
    wid = c * 16 + s
    for k in range(16):
        zbuf[pl.ds(k * 16, 16)] = jnp.zeros((16,), F32)
    for k in range(8):
        ones[pl.ds(k * 16, 16)] = jnp.full((16,), 1.0, F32)
    pltpu.sync_copy(x_hbm, xv)

    def zero_body(t, carry):
        pltpu.sync_copy(zbuf, acc.at[pl.ds(s * ZPT + t * 256, 256)])
        return carry
    lax.fori_loop(0, ZPT // 256, zero_body, 0)
    plsc.subcore_barrier()

    def chunk_body(g, carry):
        row0 = wid * RPT + g * SUBS
        pltpu.sync_copy(src_hbm.at[pl.ds(row0, SUBS)], srcb)
        pltpu.sync_copy(dst_hbm.at[pl.ds(row0, SUBS)], dstb)

        def sub_body(r, carry2):
            for k in range(8):
                sl = pl.ds(k * 16, 16)
                s16 = srcb[r, sl]
                xvals[sl] = plsc.load_gather(xv, [s16])
                d16 = dstb[r, sl] * 2
                idx1[sl] = d16
                idx2[sl] = d16 + 1
            pltpu.sync_copy(ones, acc.at[idx1], add=True)
            pltpu.sync_copy(xvals, acc.at[idx2], add=True)
            return carry2
        lax.fori_loop(0, SUBS, sub_body, 0)
        return carry
    lax.fori_loop(0, CHUNKS, chunk_body, 0)
    plsc.subcore_barrier()

    def dump_body(t, carry):
        off = s * ZPT + t * 256
        pltpu.sync_copy(acc.at[pl.ds(off, 256)], out_hbm.at[c, pl.ds(off, 256)])
        return carry
    lax.fori_loop(0, ZPT // 256, dump_body, 0)


@functools.partial(
    pl.kernel,
    out_type=jax.ShapeDtypeStruct((2, N_PAD, H), F32),
    mesh=_mesh,
    compiler_params=pltpu.CompilerParams(needs_layout_passes=False, use_tc_tiling_on_sc=False),
    scratch_types=[
        pltpu.VMEM((SUBS, LSUB), jnp.int32),
        pltpu.VMEM((SUBS, LSUB), jnp.int32),
        pltpu.VMEM((LSUB, H), F32),
        pltpu.VMEM((LSUB, H), F32),
        pltpu.VMEM((LSUB, H), F32),
        pltpu.VMEM((LSUB, H), F32),
        pltpu.VMEM((LSUB, H), F32),
        pltpu.SemaphoreType.DMA,
        pltpu.SemaphoreType.DMA,
        pltpu.SemaphoreType.DMA,
        pltpu.SemaphoreType.DMA,
        pltpu.VMEM_SHARED((N_PAD, H), F32),
    ],
)
def _sc_seg16(h_hbm, src_hbm, dst_hbm, out_hbm, srcb, dstb,
              rb0, rb1, rb2, rb3, zbuf, sm0, sm1, sm2, sm3, acc):
    rbufs = (rb0, rb1, rb2, rb3)
    sems = (sm0, sm1, sm2, sm3)
    c = lax.axis_index("c")
    s = lax.axis_index("s")
    wid = c * 16 + s
    zv = jnp.zeros((16,), F32)
    for k in range(LSUB):
        zbuf[k, :] = zv

    def zero_body(t, carry):
        pltpu.sync_copy(zbuf, acc.at[pl.ds(s * ZROWS + t * LSUB, LSUB)])
        return carry
    lax.fori_loop(0, ZROWS // LSUB, zero_body, 0)
    plsc.subcore_barrier()

    def chunk_body(g, carry):
        row0 = wid * RPT + g * SUBS
        pltpu.sync_copy(src_hbm.at[pl.ds(row0, SUBS)], srcb)
        pltpu.sync_copy(dst_hbm.at[pl.ds(row0, SUBS)], dstb)

        for p in range(3):
            pltpu.async_copy(h_hbm.at[srcb.at[p]], rbufs[p], sems[p])

        def quad_body(q, carry2):
            for p in range(4):
                r = 4 * q + p
                pltpu.make_async_copy(h_hbm.at[srcb.at[r]], rbufs[p], sems[p]).wait()

                @pl.when(r + 3 < SUBS)
                def _():
                    pltpu.async_copy(h_hbm.at[srcb.at[r + 3]],
                                     rbufs[(p + 3) % 4], sems[(p + 3) % 4])
                pltpu.sync_copy(rbufs[p], acc.at[dstb.at[r]], add=True)
            return carry2
        lax.fori_loop(0, SUBS // 4, quad_body, 0)
        return carry
    lax.fori_loop(0, CHUNKS, chunk_body, 0)
    plsc.subcore_barrier()

    def dump_body(t, carry):
        off = s * ZROWS + t * LSUB
        pltpu.sync_copy(acc.at[pl.ds(off, LSUB)], out_hbm.at[c, pl.ds(off, LSUB)])
        return carry
    lax.fori_loop(0, ZROWS // LSUB, dump_body, 0)


DW = DEP // 2
MW = 17
SUBS_D = 56
CHUNKS_D = RPT // SUBS_D


@functools.partial(
    pl.kernel,
    out_type=jax.ShapeDtypeStruct((E_PAD,), F32),
    mesh=_mesh,
    compiler_params=pltpu.CompilerParams(needs_layout_passes=False, use_tc_tiling_on_sc=False),
    scratch_types=[
        pltpu.VMEM((SUBS_D, LSUB), jnp.int32),
        pltpu.VMEM((SUBS_D, LSUB), jnp.int32),
        pltpu.VMEM((LSUB, DW), jnp.int32),
        pltpu.VMEM((LSUB, DW), jnp.int32),
        pltpu.VMEM((LSUB, DW), jnp.int32),
        pltpu.VMEM((LSUB, DW), jnp.int32),
        pltpu.VMEM((SUBS_D * LSUB,), F32),
        pltpu.VMEM((64,), jnp.int32),
        pltpu.SemaphoreType.DMA,
        pltpu.SemaphoreType.DMA,
    ],
)
def _sc_edge(a_hbm, b_hbm, wbc_hbm, src_hbm, dst_hbm, out_hbm,
             srcb, dstb, ar0, ar1, br0, br1, obuf, wv, sm0, sm1):
    c = lax.axis_index("c")
    s = lax.axis_index("s")
    wid = c * 16 + s
    iota = lax.iota(jnp.int32, 16)
    pltpu.sync_copy(wbc_hbm, wv)
    e16s = [iota + k * 16 for k in range(8)]
    abufs = (ar0, ar1)
    bbufs = (br0, br1)
    sems = (sm0, sm1)
    himask = jnp.full((16,), -65536, jnp.int32)

    def unpack(word):
        lo = plsc.bitcast(lax.shift_left(word, 16), F32)
        hi = plsc.bitcast(lax.bitwise_and(word, himask), F32)
        return lo, hi

    def compute(arows, brows, r):
        cvec = plsc.bitcast(wv[pl.ds(48, 16)], F32)
        accs = [cvec for _ in range(8)]
        mvec = iota
        dec = jnp.full((16,), MW, jnp.int32)
        for m0 in range(MW):
            wlo, whi = unpack(plsc.load_gather(wv, [mvec]))
            blo, bhi = unpack(plsc.load_gather(wv, [mvec + 24]))
            for k in range(8):
                alo, ahi = unpack(plsc.load_gather(arows, [e16s[k], mvec]))
                clo, chi = unpack(plsc.load_gather(brows, [e16s[k], mvec]))
                v0 = jnp.maximum(alo + clo + blo, 0.0)
                v1 = jnp.maximum(ahi + chi + bhi, 0.0)
                accs[k] = accs[k] + v0 * wlo + v1 * whi
            mnext = mvec + 1
            mvec = jnp.where(mnext >= dec, mnext - dec, mnext)
        for k in range(8):
            obuf[pl.ds(r * LSUB + k * 16, 16)] = accs[k]

    def chunk_body(g, carry):
        row0 = wid * RPT + g * SUBS_D
        pltpu.sync_copy(src_hbm.at[pl.ds(row0, SUBS_D)], srcb)
        pltpu.sync_copy(dst_hbm.at[pl.ds(row0, SUBS_D)], dstb)
        pltpu.async_copy(a_hbm.at[srcb.at[0]], ar0, sm0)
        pltpu.async_copy(b_hbm.at[dstb.at[0]], br0, sm0)

        def pair_body(q, carry2):
            for p in range(2):
                r = 2 * q + p
                pltpu.make_async_copy(a_hbm.at[srcb.at[r]], abufs[p], sems[p]).wait()
                pltpu.make_async_copy(b_hbm.at[dstb.at[r]], bbufs[p], sems[p]).wait()

                @pl.when(r + 1 < SUBS_D)
                def _():
                    pltpu.async_copy(a_hbm.at[srcb.at[r + 1]], abufs[1 - p], sems[1 - p])
                    pltpu.async_copy(b_hbm.at[dstb.at[r + 1]], bbufs[1 - p], sems[1 - p])
                compute(abufs[p], bbufs[p], r)
            return carry2
        lax.fori_loop(0, SUBS_D // 2, pair_body, 0)
        ebase = (wid * RPT + g * SUBS_D) * LSUB
        pltpu.sync_copy(obuf, out_hbm.at[pl.ds(ebase, SUBS_D * LSUB)])
        return carry
    lax.fori_loop(0, CHUNKS_D, chunk_body, 0)


def _tc_layer1(part, x_pad, Wl, bl, Wr):
    def body(p_ref, x_ref, wl_ref, bl_ref, wr_ref, h_ref, d_ref):
        p = p_ref[...]
        cnt = p[0, :, 0] + p[1, :, 0]
        agg = p[0, :, 1] + p[1, :, 1]
        dcl = jnp.maximum(cnt, 1.0)
        mean = (agg / dcl)[:, None]
        h = mean * wl_ref[...] + bl_ref[...] + x_ref[...] * wr_ref[...]
        h_ref[...] = jnp.maximum(h, 0.0)
        d_ref[...] = dcl[:, None]

    return pl.pallas_call(
        body,
        grid=(GRID_N,),
        in_specs=[
            pl.BlockSpec((2, BN, 2), lambda i: (0, i, 0)),
            pl.BlockSpec((BN, 1), lambda i: (i, 0)),
            pl.BlockSpec((1, H), lambda i: (0, 0)),
            pl.BlockSpec((1, H), lambda i: (0, 0)),
            pl.BlockSpec((1, H), lambda i: (0, 0)),
        ],
        out_specs=[
            pl.BlockSpec((BN, H), lambda i: (i, 0)),
            pl.BlockSpec((BN, 1), lambda i: (i, 0)),
        ],
        out_shape=[
            jax.ShapeDtypeStruct((N_PAD, H), F32),
            jax.ShapeDtypeStruct((N_PAD, 1), F32),
        ],
    )(part, x_pad, Wl, bl, Wr)


def _tc_layer(part, deg, h_prev, Wl, bl, Wr):
    def body(p_ref, d_ref, h_ref, wl_ref, bl_ref, wr_ref, o_ref):
        p = p_ref[...]
        mean = (p[0] + p[1]) / d_ref[...]
        o = (jnp.dot(mean, wl_ref[...], preferred_element_type=F32)
             + bl_ref[...]
             + jnp.dot(h_ref[...], wr_ref[...], preferred_element_type=F32))
        o_ref[...] = jnp.maximum(o, 0.0)

    return pl.pallas_call(
        body,
        grid=(GRID_N,),
        in_specs=[
            pl.BlockSpec((2, BN, H), lambda i: (0, i, 0)),
            pl.BlockSpec((BN, 1), lambda i: (i, 0)),
            pl.BlockSpec((BN, H), lambda i: (i, 0)),
            pl.BlockSpec((H, H), lambda i: (0, 0)),
            pl.BlockSpec((1, H), lambda i: (0, 0)),
            pl.BlockSpec((H, H), lambda i: (0, 0)),
        ],
        out_specs=pl.BlockSpec((BN, H), lambda i: (i, 0)),
        out_shape=jax.ShapeDtypeStruct((N_PAD, H), F32),
    )(part, deg, h_prev, Wl, bl, Wr)


def _tc_final(part, deg, h_prev, x_pad, Wl, bl, Wr, nW, nb, ew0, ew1, ews, ewd):
    def body(p_ref, d_ref, h_ref, x_ref, wl_ref, bl_ref, wr_ref,
             nw_ref, nb_ref, e0_ref, e1_ref, es_ref, ed_ref,
             no_ref, a_ref, b_ref):
        p = p_ref[...]
        mean = (p[0] + p[1]) / d_ref[...]
        h3 = (jnp.dot(mean, wl_ref[...], preferred_element_type=F32)
              + bl_ref[...]
              + jnp.dot(h_ref[...], wr_ref[...], preferred_element_type=F32))
        h3 = jnp.maximum(h3, 0.0)
        no_ref[...] = jnp.dot(h3, nw_ref[...], preferred_element_type=F32) + nb_ref[...]
        xb = x_ref[...]
        a_ref[...] = xb * e0_ref[...] + jnp.dot(h3, es_ref[...], preferred_element_type=F32)
        b_ref[...] = xb * e1_ref[...] + jnp.dot(h3, ed_ref[...], preferred_element_type=F32)

    return pl.pallas_call(
        body,
        grid=(GRID_N,),
        in_specs=[
            pl.BlockSpec((2, BN, H), lambda i: (0, i, 0)),
            pl.BlockSpec((BN, 1), lambda i: (i, 0)),
            pl.BlockSpec((BN, H), lambda i: (i, 0)),
            pl.BlockSpec((BN, 1), lambda i: (i, 0)),
            pl.BlockSpec((H, H), lambda i: (0, 0)),
            pl.BlockSpec((1, H), lambda i: (0, 0)),
            pl.BlockSpec((H, H), lambda i: (0, 0)),
            pl.BlockSpec((H, 1), lambda i: (0, 0)),
            pl.BlockSpec((1, 1), lambda i: (0, 0)),
            pl.BlockSpec((1, DEP), lambda i: (0, 0)),
            pl.BlockSpec((1, DEP), lambda i: (0, 0)),
            pl.BlockSpec((H, DEP), lambda i: (0, 0)),
            pl.BlockSpec((H, DEP), lambda i: (0, 0)),
        ],
        out_specs=[
            pl.BlockSpec((BN, 1), lambda i: (i, 0)),
            pl.BlockSpec((BN, DEP), lambda i: (i, 0)),
            pl.BlockSpec((BN, DEP), lambda i: (i, 0)),
        ],
        out_shape=[
            jax.ShapeDtypeStruct((N_PAD, 1), F32),
            jax.ShapeDtypeStruct((N_PAD, DEP), F32),
            jax.ShapeDtypeStruct((N_PAD, DEP), F32),
        ],
    )(part, deg, h_prev, x_pad, Wl, bl, Wr, nW, nb, ew0, ew1, ews, ewd)


def kernel(x, edge_index,
           conv1_Wl, conv1_bl, conv1_Wr,
           conv2_Wl, conv2_bl, conv2_Wr,
           conv3_Wl, conv3_bl, conv3_Wr,
           edge_W, edge_b, node_W, node_b, ecls_W, ecls_b):
    src = edge_index[0]
    dst = edge_index[1]
    pad_e = E_PAD - E
    src2 = jnp.concatenate([src, jnp.zeros((pad_e,), jnp.int32)]).reshape(ROWS, LSUB)
    dst2 = jnp.concatenate([dst, jnp.full((pad_e,), N, jnp.int32)]).reshape(ROWS, LSUB)
    x_pad = jnp.concatenate([x, jnp.zeros((N_PAD - N, 1), F32)])
    x_flat = x_pad[:, 0]

    part1 = _sc_deg_agg(x_flat, src2, dst2).reshape(2, N_PAD, 2)
    h1, deg = _tc_layer1(part1, x_pad, conv1_Wl, conv1_bl.reshape(1, H), conv1_Wr)
    part2 = _sc_seg16(h1, src2, dst2)
    h2 = _tc_layer(part2, deg, h1, conv2_Wl, conv2_bl.reshape(1, H), conv2_Wr)
    part3 = _sc_seg16(h2, src2, dst2)
    ewp = jnp.pad(edge_W, ((0, 0), (0, DEP - DE)))
    node_full, atab, btab = _tc_final(
        part3, deg, h2, x_pad,
        conv3_Wl, conv3_bl.reshape(1, H), conv3_Wr,
        node_W, node_b.reshape(1, 1),
        ewp[0].reshape(1, DEP), ewp[1].reshape(1, DEP),
        ewp[2:2 + H], ewp[2 + H:2 + 2 * H])

    bf16 = jnp.bfloat16
    atab_p = lax.bitcast_convert_type(
        atab.astype(bf16).reshape(N_PAD, DW, 2), jnp.int32)
    btab_p = lax.bitcast_convert_type(
        btab.astype(bf16).reshape(N_PAD, DW, 2), jnp.int32)
    wpack = lax.bitcast_convert_type(
        jnp.pad(ecls_W[:, 0], (0, DEP - DE)).astype(bf16).reshape(DW, 2), jnp.int32)
    bpack = lax.bitcast_convert_type(
        jnp.pad(edge_b, (0, DEP - DE)).astype(bf16).reshape(DW, 2), jnp.int32)
    cbits = lax.bitcast_convert_type(jnp.broadcast_to(ecls_b, (16,)), jnp.int32)
    wbc = jnp.concatenate([wpack, bpack, cbits])
    eout = _sc_edge(atab_p, btab_p, wbc, src2, dst2)

    return (node_full[:N], eout[:E][:, None])
